# baseline (speedup 1.0000x reference)
import functools

import jax
import jax.numpy as jnp
from jax import lax
from jax.experimental import pallas as pl
from jax.experimental.pallas import tpu as pltpu
from jax.experimental.pallas import tpu_sc as plsc

NC = 2
NS = 16
NW = NC * NS
CHUNK = 320
SPLITS = (128, 128, 64)
TC = 512
LANES = 128


@functools.lru_cache(maxsize=None)
def _transpose_widen_call(dim, v):
    blocks = v // TC
    per_w = blocks // NW
    rem0 = NW * per_w
    rem = blocks - rem0
    tail = v - blocks * TC
    assert tail > 0
    vw = blocks * TC + 128

    mesh = plsc.VectorSubcoreMesh(core_axis_name="c", subcore_axis_name="s")

    @functools.partial(
        pl.kernel,
        mesh=mesh,
        out_type=jax.ShapeDtypeStruct((vw * LANES,), jnp.float32),
        compiler_params=pltpu.CompilerParams(needs_layout_passes=False),
        scratch_types=[
            pltpu.VMEM((dim, TC), jnp.float32),
            pltpu.VMEM((TC * LANES,), jnp.float32),
        ],
    )
    def k(tbl_t_hbm, tail_hbm, out_hbm, vin, vout):
        wid = lax.axis_index("s") * NC + lax.axis_index("c")
        riota = lax.iota(jnp.int32, 16) * LANES

        def transpose_cols(ncols):
            def kbody(kk, carry):
                base = kk * (16 * LANES)
                for d in range(dim):
                    val = vin[d, pl.ds(kk * 16, 16)]
                    plsc.store_scatter(vout, [riota + (base + d)], val)
                return carry

            lax.fori_loop(0, ncols // 16, kbody, 0)

        def do_block(c0):
            pltpu.sync_copy(tbl_t_hbm.at[:, pl.ds(c0, TC)], vin)
            transpose_cols(TC)
            pltpu.sync_copy(
                vout, out_hbm.at[pl.ds(c0 * LANES, TC * LANES)]
            )

        def body(bi, carry):
            do_block((wid * per_w + bi) * TC)
            return carry

        lax.fori_loop(0, per_w, body, 0)

        @pl.when(wid < rem)
        def _():
            do_block((rem0 + wid) * TC)

        @pl.when(wid == NW - 1)
        def _():
            pltpu.sync_copy(tail_hbm, vin.at[:, pl.ds(0, 128)])
            transpose_cols(128)
            pltpu.sync_copy(
                vout.at[pl.ds(0, 128 * LANES)],
                out_hbm.at[pl.ds(blocks * TC * LANES, 128 * LANES)],
            )

    return k, vw


@functools.lru_cache(maxsize=None)
def _gather_call(n_rows, dimp):
    rows_per_w = n_rows // NW
    steps = rows_per_w // CHUNK
    pairs = steps // 2

    mesh = plsc.VectorSubcoreMesh(core_axis_name="c", subcore_axis_name="s")

    @functools.partial(
        pl.kernel,
        mesh=mesh,
        out_type=jax.ShapeDtypeStruct((n_rows, dimp), jnp.float32),
        scratch_types=[
            pltpu.VMEM((CHUNK,), jnp.int32),
            pltpu.VMEM((CHUNK,), jnp.int32),
            pltpu.VMEM((CHUNK, dimp), jnp.float32),
            pltpu.VMEM((CHUNK, dimp), jnp.float32),
            pltpu.SemaphoreType.DMA,
            pltpu.SemaphoreType.DMA,
            pltpu.SemaphoreType.DMA,
        ],
    )
    def k(table_hbm, idx_hbm, out_hbm, idx0, idx1, buf0, buf1, g0s, g1s, wsem):
        wid = lax.axis_index("s") * NC + lax.axis_index("c")
        row0 = wid * rows_per_w

        def fire(c, idx_v, buf, sem):
            pltpu.sync_copy(idx_hbm.at[pl.ds(row0 + c * CHUNK, CHUNK)], idx_v)
            off = 0
            for n in SPLITS:
                pltpu.async_copy(
                    table_hbm.at[idx_v.at[pl.ds(off, n)]],
                    buf.at[pl.ds(off, n)],
                    sem,
                )
                off += n

        def drain_gathers(buf, sem):
            off = 0
            for n in SPLITS:
                pltpu.make_async_copy(
                    table_hbm.at[pl.ds(0, n)],
                    buf.at[pl.ds(off, n)],
                    sem,
                ).wait()
                off += n

        def writeback(c, buf):
            return pltpu.async_copy(
                buf, out_hbm.at[pl.ds(row0 + c * CHUNK, CHUNK)], wsem
            )

        def drain_writebacks():
            pltpu.make_async_copy(
                buf0, out_hbm.at[pl.ds(row0, CHUNK)], wsem
            ).wait()
            pltpu.make_async_copy(
                buf1, out_hbm.at[pl.ds(row0, CHUNK)], wsem
            ).wait()

        def body(t, carry):
            c0 = 2 * t
            c1 = c0 + 1

            @pl.when(t > 0)
            def _():
                drain_writebacks()

            fire(c0, idx0, buf0, g0s)
            fire(c1, idx1, buf1, g1s)
            drain_gathers(buf0, g0s)
            writeback(c0, buf0)
            drain_gathers(buf1, g1s)
            writeback(c1, buf1)
            return carry

        lax.fori_loop(0, pairs, body, 0)
        drain_writebacks()

    return k


def kernel(x, table):
    b, l = x.shape
    n = b * l
    v, dim = table.shape
    tw, vw = _transpose_widen_call(dim, v)
    nfull = (v // TC) * TC
    tail_t = jnp.pad(table[nfull:].T, ((0, 0), (0, 128 - (v - nfull))))
    table_w = tw(table.T, tail_t).reshape(vw, LANES)
    idx_flat = x.reshape(n).astype(jnp.int32)
    out_w = _gather_call(n, LANES)(table_w, idx_flat)
    return out_w[:, :dim].reshape(b, l, dim)


# speedup vs baseline: 1.9584x; 1.9584x over previous
import functools

import jax
import jax.numpy as jnp
from jax import lax
from jax.experimental import pallas as pl
from jax.experimental.pallas import tpu as pltpu
from jax.experimental.pallas import tpu_sc as plsc

NC = 2
NS = 16
NW = NC * NS
CHUNK = 320
SPLITS = (128, 128, 64)


@functools.lru_cache(maxsize=None)
def _gather_call(n_rows, dimp):
    rows_per_w = n_rows // NW
    steps = rows_per_w // CHUNK
    pairs = steps // 2

    mesh = plsc.VectorSubcoreMesh(core_axis_name="c", subcore_axis_name="s")

    @functools.partial(
        pl.kernel,
        mesh=mesh,
        out_type=jax.ShapeDtypeStruct((n_rows, dimp), jnp.float32),
        scratch_types=[
            pltpu.VMEM((CHUNK,), jnp.int32),
            pltpu.VMEM((CHUNK,), jnp.int32),
            pltpu.VMEM((CHUNK, dimp), jnp.float32),
            pltpu.VMEM((CHUNK, dimp), jnp.float32),
            pltpu.SemaphoreType.DMA,
            pltpu.SemaphoreType.DMA,
            pltpu.SemaphoreType.DMA,
        ],
    )
    def k(table_hbm, idx_hbm, out_hbm, idx0, idx1, buf0, buf1, g0s, g1s, wsem):
        wid = lax.axis_index("s") * NC + lax.axis_index("c")
        row0 = wid * rows_per_w

        def fire(c, idx_v, buf, sem):
            pltpu.sync_copy(idx_hbm.at[pl.ds(row0 + c * CHUNK, CHUNK)], idx_v)
            off = 0
            for n in SPLITS:
                pltpu.async_copy(
                    table_hbm.at[idx_v.at[pl.ds(off, n)]],
                    buf.at[pl.ds(off, n)],
                    sem,
                )
                off += n

        def drain_gathers(buf, sem):
            off = 0
            for n in SPLITS:
                pltpu.make_async_copy(
                    table_hbm.at[pl.ds(0, n)],
                    buf.at[pl.ds(off, n)],
                    sem,
                ).wait()
                off += n

        def writeback(c, buf):
            return pltpu.async_copy(
                buf, out_hbm.at[pl.ds(row0 + c * CHUNK, CHUNK)], wsem
            )

        def drain_writebacks():
            pltpu.make_async_copy(
                buf0, out_hbm.at[pl.ds(row0, CHUNK)], wsem
            ).wait()
            pltpu.make_async_copy(
                buf1, out_hbm.at[pl.ds(row0, CHUNK)], wsem
            ).wait()

        def body(t, carry):
            c0 = 2 * t
            c1 = c0 + 1

            @pl.when(t > 0)
            def _():
                drain_writebacks()

            fire(c0, idx0, buf0, g0s)
            fire(c1, idx1, buf1, g1s)
            drain_gathers(buf0, g0s)
            writeback(c0, buf0)
            drain_gathers(buf1, g1s)
            writeback(c1, buf1)
            return carry

        lax.fori_loop(0, pairs, body, 0)
        drain_writebacks()

    return k


def kernel(x, table):
    b, l = x.shape
    n = b * l
    v, dim = table.shape
    table_w = jnp.pad(table, ((0, 0), (0, 64)))
    idx_flat = x.reshape(n).astype(jnp.int32)
    out_w = _gather_call(n, 2 * dim)(table_w, idx_flat)
    return out_w[:, :dim].reshape(b, l, dim)
